# baseline (speedup 1.0000x reference)
import functools

import jax
import jax.numpy as jnp
from jax import lax
from jax.experimental import pallas as pl
from jax.experimental.pallas import tpu as pltpu
from jax.experimental.pallas import tpu_sc as plsc

N_NODES = 10000
N_EDGES = 160000
NODE_IN = 128
EDGE_IN = 16
HIDDEN = 16

GRP = 128
N_GROUPS = N_EDGES // GRP
NW = 32
GROUPS_PER_W = -(-N_GROUPS // NW)
N_PAD = 10240
ROWS_PER_TILE = N_PAD // 16
RCH = ROWS_PER_TILE // GRP

EDGE_BLK = 640
MW = 128


_sc_mesh = plsc.VectorSubcoreMesh(core_axis_name="c", subcore_axis_name="s")


@functools.partial(
    pl.kernel,
    mesh=_sc_mesh,
    out_type=jax.ShapeDtypeStruct((N_EDGES, NODE_IN), jnp.float32),
    scratch_types=[
        pltpu.VMEM((GRP,), jnp.int32),
        pltpu.VMEM((GRP, NODE_IN), jnp.float32),
        pltpu.SemaphoreType.DMA,
    ],
)
def _sc_gather(table_hbm, idx_hbm, out_hbm, idx_v, rows_v, sem):
    wid = lax.axis_index("s") * 2 + lax.axis_index("c")

    def body(g, carry):
        gid = g * NW + wid

        @pl.when(gid < N_GROUPS)
        def _():
            base = gid * GRP
            pltpu.sync_copy(idx_hbm.at[pl.ds(base, GRP)], idx_v)
            pltpu.async_copy(table_hbm.at[idx_v], rows_v, sem).wait()
            pltpu.sync_copy(rows_v, out_hbm.at[pl.ds(base, GRP)])

        return carry

    lax.fori_loop(0, GROUPS_PER_W, body, 0)


@functools.partial(
    pl.kernel,
    mesh=_sc_mesh,
    out_type=jax.ShapeDtypeStruct((2, N_PAD, MW), jnp.float32),
    scratch_types=[
        pltpu.VMEM((GRP, MW), jnp.float32),
        pltpu.VMEM((RCH, GRP), jnp.int32),
        pltpu.VMEM((1, GRP), jnp.int32),
        pltpu.VMEM((GRP, MW), jnp.float32),
        pltpu.VMEM_SHARED((N_PAD, MW), jnp.float32),
        pltpu.SemaphoreType.DMA,
    ],
)
def _sc_scatter(m_hbm, dst_hbm, rowidx_hbm, zeros_hbm, out_hbm,
                stage_v, ridx_v, idx_v, rows_v, acc_sh, sem):
    cid = lax.axis_index("c")
    sid = lax.axis_index("s")
    wid = sid * 2 + cid
    row0 = sid * ROWS_PER_TILE

    pltpu.sync_copy(rowidx_hbm.at[sid], ridx_v)
    pltpu.sync_copy(zeros_hbm, stage_v)
    for j in range(RCH):
        pltpu.sync_copy(stage_v, acc_sh.at[ridx_v.at[j]])
    plsc.subcore_barrier()

    def body(g, carry):
        gid = g * NW + wid

        @pl.when(gid < N_GROUPS)
        def _():
            pltpu.sync_copy(dst_hbm.at[pl.ds(gid, 1)], idx_v)
            pltpu.sync_copy(m_hbm.at[pl.ds(gid * GRP, GRP)], rows_v)
            pltpu.sync_copy(rows_v, acc_sh.at[idx_v.at[0]], add=True)

        return carry

    lax.fori_loop(0, GROUPS_PER_W, body, 0)
    plsc.subcore_barrier()

    for j in range(RCH):
        pltpu.async_copy(acc_sh.at[ridx_v.at[j]], stage_v, sem).wait()
        pltpu.sync_copy(stage_v, out_hbm.at[cid, pl.ds(row0 + j * GRP, GRP)])


def _edge_body(s_ref, h_ref, wet_ref, be_ref, out_ref):
    s = s_ref[...].astype(jnp.bfloat16)
    h = h_ref[...]
    a = jnp.dot(s, wet_ref[...].astype(jnp.bfloat16),
                preferred_element_type=jnp.float32)
    e = jnp.maximum(a + be_ref[...], 0.0)
    cols = []
    for k in range(HIDDEN):
        seg = e[:, k * NODE_IN:(k + 1) * NODE_IN] * h
        cols.append(jnp.sum(seg, axis=1, keepdims=True))
    ones = jnp.ones((EDGE_BLK, 1), jnp.float32)
    pad = jnp.zeros((EDGE_BLK, MW - HIDDEN - 1), jnp.float32)
    out_ref[...] = jnp.concatenate(cols + [ones, pad], axis=1)


def _edge_messages(s, h_src, wet, be2):
    grid = N_EDGES // EDGE_BLK
    return pl.pallas_call(
        _edge_body,
        grid=(grid,),
        in_specs=[
            pl.BlockSpec((EDGE_BLK, EDGE_IN), lambda i: (i, 0)),
            pl.BlockSpec((EDGE_BLK, NODE_IN), lambda i: (i, 0)),
            pl.BlockSpec((EDGE_IN, HIDDEN * NODE_IN), lambda i: (0, 0)),
            pl.BlockSpec((1, HIDDEN * NODE_IN), lambda i: (0, 0)),
        ],
        out_specs=pl.BlockSpec((EDGE_BLK, MW), lambda i: (i, 0)),
        out_shape=jax.ShapeDtypeStruct((N_EDGES, MW), jnp.float32),
    )(s, h_src, wet, be2)


def _final_body(nf_ref, p_ref, wn1_ref, wn2_ref, bn_ref, out_ref):
    nf = nf_ref[...]
    p = p_ref[...]
    ps = p[0, :N_NODES, :HIDDEN + 1] + p[1, :N_NODES, :HIDDEN + 1]
    msum = ps[:, :HIDDEN]
    cnt = ps[:, HIDDEN:HIDDEN + 1]
    h_neigh = msum / jnp.maximum(cnt, 1.0)
    a = jnp.dot(nf, wn1_ref[...], preferred_element_type=jnp.float32)
    b = jnp.dot(h_neigh, wn2_ref[...], preferred_element_type=jnp.float32)
    out_ref[...] = jnp.maximum(a + b + bn_ref[...], 0.0)


def _finish(nf, partials, wn1, wn2, bn2):
    return pl.pallas_call(
        _final_body,
        out_shape=jax.ShapeDtypeStruct((N_NODES, HIDDEN), jnp.float32),
    )(nf, partials, wn1, wn2, bn2)


@jax.jit
def kernel(node_features, edge_index, static_edge_features, We, be, Wn, bn):
    src = edge_index[0].astype(jnp.int32)
    dst = edge_index[1].astype(jnp.int32)
    wet = We.T
    be2 = be.reshape(1, HIDDEN * NODE_IN)
    wn1 = Wn[:, :NODE_IN].T
    wn2 = Wn[:, NODE_IN:].T
    bn2 = bn.reshape(1, HIDDEN)
    zeros = jnp.zeros((GRP, MW), jnp.float32)
    rowidx = (jnp.arange(16, dtype=jnp.int32)[:, None] * ROWS_PER_TILE
              + jnp.arange(ROWS_PER_TILE, dtype=jnp.int32)
              ).reshape(16, RCH, GRP)
    dst2 = dst.reshape(N_GROUPS, GRP)

    h_src = _sc_gather(node_features, src)
    m_ext = _edge_messages(static_edge_features, h_src, wet, be2)
    partials = _sc_scatter(m_ext, dst2, rowidx, zeros)
    return _finish(node_features, partials, wn1, wn2, bn2)


# speedup vs baseline: 1.0456x; 1.0456x over previous
import functools

import jax
import jax.numpy as jnp
from jax import lax
from jax.experimental import pallas as pl
from jax.experimental.pallas import tpu as pltpu
from jax.experimental.pallas import tpu_sc as plsc

N_NODES = 10000
N_EDGES = 160000
NODE_IN = 128
EDGE_IN = 16
HIDDEN = 16

GRP = 128
N_GROUPS = N_EDGES // GRP
NW = 32
GROUPS_PER_W = -(-N_GROUPS // NW)
N_PAD = 10240
ROWS_PER_TILE = N_PAD // 16
RCH = ROWS_PER_TILE // GRP

EDGE_BLK = 1280
MW = 128


_sc_mesh = plsc.VectorSubcoreMesh(core_axis_name="c", subcore_axis_name="s")


@functools.partial(
    pl.kernel,
    mesh=_sc_mesh,
    out_type=jax.ShapeDtypeStruct((N_EDGES, NODE_IN), jnp.float32),
    scratch_types=[
        pltpu.VMEM((GRP,), jnp.int32),
        pltpu.VMEM((GRP, NODE_IN), jnp.float32),
        pltpu.SemaphoreType.DMA,
    ],
)
def _sc_gather(table_hbm, idx_hbm, out_hbm, idx_v, rows_v, sem):
    wid = lax.axis_index("s") * 2 + lax.axis_index("c")

    def body(g, carry):
        gid = g * NW + wid

        @pl.when(gid < N_GROUPS)
        def _():
            base = gid * GRP
            pltpu.sync_copy(idx_hbm.at[pl.ds(base, GRP)], idx_v)
            pltpu.async_copy(table_hbm.at[idx_v], rows_v, sem).wait()
            pltpu.sync_copy(rows_v, out_hbm.at[pl.ds(base, GRP)])

        return carry

    lax.fori_loop(0, GROUPS_PER_W, body, 0)


@functools.partial(
    pl.kernel,
    mesh=_sc_mesh,
    out_type=jax.ShapeDtypeStruct((2, N_PAD, MW), jnp.float32),
    scratch_types=[
        pltpu.VMEM((GRP, MW), jnp.float32),
        pltpu.VMEM((RCH, GRP), jnp.int32),
        pltpu.VMEM((1, GRP), jnp.int32),
        pltpu.VMEM((GRP, MW), jnp.float32),
        pltpu.VMEM_SHARED((N_PAD, MW), jnp.float32),
        pltpu.SemaphoreType.DMA,
    ],
)
def _sc_scatter(m_hbm, dst_hbm, rowidx_hbm, zeros_hbm, out_hbm,
                stage_v, ridx_v, idx_v, rows_v, acc_sh, sem):
    cid = lax.axis_index("c")
    sid = lax.axis_index("s")
    wid = sid * 2 + cid
    row0 = sid * ROWS_PER_TILE

    pltpu.sync_copy(rowidx_hbm.at[sid], ridx_v)
    pltpu.sync_copy(zeros_hbm, stage_v)
    for j in range(RCH):
        pltpu.sync_copy(stage_v, acc_sh.at[ridx_v.at[j]])
    plsc.subcore_barrier()

    def body(g, carry):
        gid = g * NW + wid

        @pl.when(gid < N_GROUPS)
        def _():
            pltpu.sync_copy(dst_hbm.at[pl.ds(gid, 1)], idx_v)
            pltpu.sync_copy(m_hbm.at[pl.ds(gid * GRP, GRP)], rows_v)
            pltpu.sync_copy(rows_v, acc_sh.at[idx_v.at[0]], add=True)

        return carry

    lax.fori_loop(0, GROUPS_PER_W, body, 0)
    plsc.subcore_barrier()

    for j in range(RCH):
        pltpu.async_copy(acc_sh.at[ridx_v.at[j]], stage_v, sem).wait()
        pltpu.sync_copy(stage_v, out_hbm.at[cid, pl.ds(row0 + j * GRP, GRP)])


def _edge_body(s_ref, h_ref, wet_ref, be_ref, out_ref):
    s = s_ref[...]
    h = h_ref[...]
    a = jnp.dot(s, wet_ref[...], preferred_element_type=jnp.float32)
    e = jnp.maximum(a + be_ref[...], 0.0)
    cols = []
    for k in range(HIDDEN):
        seg = e[:, k * NODE_IN:(k + 1) * NODE_IN] * h
        cols.append(jnp.sum(seg, axis=1, keepdims=True))
    ones = jnp.ones((EDGE_BLK, 1), jnp.float32)
    pad = jnp.zeros((EDGE_BLK, MW - HIDDEN - 1), jnp.float32)
    out_ref[...] = jnp.concatenate(cols + [ones, pad], axis=1)


def _edge_messages(s, h_src, wet, be2):
    grid = N_EDGES // EDGE_BLK
    return pl.pallas_call(
        _edge_body,
        grid=(grid,),
        in_specs=[
            pl.BlockSpec((EDGE_BLK, EDGE_IN), lambda i: (i, 0)),
            pl.BlockSpec((EDGE_BLK, NODE_IN), lambda i: (i, 0)),
            pl.BlockSpec((EDGE_IN, HIDDEN * NODE_IN), lambda i: (0, 0)),
            pl.BlockSpec((1, HIDDEN * NODE_IN), lambda i: (0, 0)),
        ],
        out_specs=pl.BlockSpec((EDGE_BLK, MW), lambda i: (i, 0)),
        out_shape=jax.ShapeDtypeStruct((N_EDGES, MW), jnp.float32),
    )(s, h_src, wet, be2)


def _final_body(nf_ref, p_ref, wn1_ref, wn2_ref, bn_ref, out_ref):
    nf = nf_ref[...]
    p = p_ref[...]
    ps = p[0, :N_NODES, :HIDDEN + 1] + p[1, :N_NODES, :HIDDEN + 1]
    msum = ps[:, :HIDDEN]
    cnt = ps[:, HIDDEN:HIDDEN + 1]
    h_neigh = msum / jnp.maximum(cnt, 1.0)
    a = jnp.dot(nf, wn1_ref[...], preferred_element_type=jnp.float32)
    b = jnp.dot(h_neigh, wn2_ref[...], preferred_element_type=jnp.float32)
    out_ref[...] = jnp.maximum(a + b + bn_ref[...], 0.0)


def _finish(nf, partials, wn1, wn2, bn2):
    return pl.pallas_call(
        _final_body,
        out_shape=jax.ShapeDtypeStruct((N_NODES, HIDDEN), jnp.float32),
    )(nf, partials, wn1, wn2, bn2)


@jax.jit
def kernel(node_features, edge_index, static_edge_features, We, be, Wn, bn):
    src = edge_index[0].astype(jnp.int32)
    dst = edge_index[1].astype(jnp.int32)
    wet = We.T
    be2 = be.reshape(1, HIDDEN * NODE_IN)
    wn1 = Wn[:, :NODE_IN].T
    wn2 = Wn[:, NODE_IN:].T
    bn2 = bn.reshape(1, HIDDEN)
    zeros = jnp.zeros((GRP, MW), jnp.float32)
    rowidx = (jnp.arange(16, dtype=jnp.int32)[:, None] * ROWS_PER_TILE
              + jnp.arange(ROWS_PER_TILE, dtype=jnp.int32)
              ).reshape(16, RCH, GRP)
    dst2 = dst.reshape(N_GROUPS, GRP)

    h_src = _sc_gather(node_features, src)
    m_ext = _edge_messages(static_edge_features, h_src, wet, be2)
    partials = _sc_scatter(m_ext, dst2, rowidx, zeros)
    return _finish(node_features, partials, wn1, wn2, bn2)


# speedup vs baseline: 1.0639x; 1.0175x over previous
import functools

import jax
import jax.numpy as jnp
from jax import lax
from jax.experimental import pallas as pl
from jax.experimental.pallas import tpu as pltpu
from jax.experimental.pallas import tpu_sc as plsc

N_NODES = 10000
N_EDGES = 160000
NODE_IN = 128
EDGE_IN = 16
HIDDEN = 16

GRP = 128
N_GROUPS = N_EDGES // GRP
NW = 32
GROUPS_PER_W = -(-N_GROUPS // NW)
N_PAD = 10240
ROWS_PER_TILE = N_PAD // 16
RCH = ROWS_PER_TILE // GRP

EDGE_BLK = 2000
MW = 128


_sc_mesh = plsc.VectorSubcoreMesh(core_axis_name="c", subcore_axis_name="s")


@functools.partial(
    pl.kernel,
    mesh=_sc_mesh,
    out_type=jax.ShapeDtypeStruct((N_EDGES, NODE_IN), jnp.float32),
    scratch_types=[
        pltpu.VMEM((GRP,), jnp.int32),
        pltpu.VMEM((GRP, NODE_IN), jnp.float32),
        pltpu.SemaphoreType.DMA,
    ],
)
def _sc_gather(table_hbm, idx_hbm, out_hbm, idx_v, rows_v, sem):
    wid = lax.axis_index("s") * 2 + lax.axis_index("c")

    def body(g, carry):
        gid = g * NW + wid

        @pl.when(gid < N_GROUPS)
        def _():
            base = gid * GRP
            pltpu.sync_copy(idx_hbm.at[pl.ds(base, GRP)], idx_v)
            pltpu.async_copy(table_hbm.at[idx_v], rows_v, sem).wait()
            pltpu.sync_copy(rows_v, out_hbm.at[pl.ds(base, GRP)])

        return carry

    lax.fori_loop(0, GROUPS_PER_W, body, 0)


@functools.partial(
    pl.kernel,
    mesh=_sc_mesh,
    out_type=jax.ShapeDtypeStruct((2, N_PAD, MW), jnp.float32),
    scratch_types=[
        pltpu.VMEM((GRP, MW), jnp.float32),
        pltpu.VMEM((RCH, GRP), jnp.int32),
        pltpu.VMEM((1, GRP), jnp.int32),
        pltpu.VMEM((GRP, MW), jnp.float32),
        pltpu.VMEM_SHARED((N_PAD, MW), jnp.float32),
        pltpu.SemaphoreType.DMA,
    ],
)
def _sc_scatter(m_hbm, dst_hbm, rowidx_hbm, zeros_hbm, out_hbm,
                stage_v, ridx_v, idx_v, rows_v, acc_sh, sem):
    cid = lax.axis_index("c")
    sid = lax.axis_index("s")
    wid = sid * 2 + cid
    row0 = sid * ROWS_PER_TILE

    pltpu.sync_copy(rowidx_hbm.at[sid], ridx_v)
    pltpu.sync_copy(zeros_hbm, stage_v)
    for j in range(RCH):
        pltpu.sync_copy(stage_v, acc_sh.at[ridx_v.at[j]])
    plsc.subcore_barrier()

    def body(g, carry):
        gid = g * NW + wid

        @pl.when(gid < N_GROUPS)
        def _():
            pltpu.sync_copy(dst_hbm.at[pl.ds(gid, 1)], idx_v)
            pltpu.sync_copy(m_hbm.at[pl.ds(gid * GRP, GRP)], rows_v)
            pltpu.sync_copy(rows_v, acc_sh.at[idx_v.at[0]], add=True)

        return carry

    lax.fori_loop(0, GROUPS_PER_W, body, 0)
    plsc.subcore_barrier()

    for j in range(RCH):
        pltpu.async_copy(acc_sh.at[ridx_v.at[j]], stage_v, sem).wait()
        pltpu.sync_copy(stage_v, out_hbm.at[cid, pl.ds(row0 + j * GRP, GRP)])


def _edge_body(s_ref, h_ref, wet_ref, be_ref, out_ref):
    s = s_ref[...]
    h = h_ref[...]
    a = jnp.dot(s, wet_ref[...], preferred_element_type=jnp.float32)
    e = jnp.maximum(a + be_ref[...], 0.0)
    cols = []
    for k in range(HIDDEN):
        seg = e[:, k * NODE_IN:(k + 1) * NODE_IN] * h
        cols.append(jnp.sum(seg, axis=1, keepdims=True))
    ones = jnp.ones((EDGE_BLK, 1), jnp.float32)
    pad = jnp.zeros((EDGE_BLK, MW - HIDDEN - 1), jnp.float32)
    out_ref[...] = jnp.concatenate(cols + [ones, pad], axis=1)


def _edge_messages(s, h_src, wet, be2):
    grid = N_EDGES // EDGE_BLK
    return pl.pallas_call(
        _edge_body,
        grid=(grid,),
        in_specs=[
            pl.BlockSpec((EDGE_BLK, EDGE_IN), lambda i: (i, 0)),
            pl.BlockSpec((EDGE_BLK, NODE_IN), lambda i: (i, 0)),
            pl.BlockSpec((EDGE_IN, HIDDEN * NODE_IN), lambda i: (0, 0)),
            pl.BlockSpec((1, HIDDEN * NODE_IN), lambda i: (0, 0)),
        ],
        out_specs=pl.BlockSpec((EDGE_BLK, MW), lambda i: (i, 0)),
        out_shape=jax.ShapeDtypeStruct((N_EDGES, MW), jnp.float32),
    )(s, h_src, wet, be2)


def _final_body(nf_ref, p_ref, wn1_ref, wn2_ref, bn_ref, out_ref):
    nf = nf_ref[...]
    p = p_ref[...]
    ps = p[0, :N_NODES, :HIDDEN + 1] + p[1, :N_NODES, :HIDDEN + 1]
    msum = ps[:, :HIDDEN]
    cnt = ps[:, HIDDEN:HIDDEN + 1]
    h_neigh = msum / jnp.maximum(cnt, 1.0)
    a = jnp.dot(nf, wn1_ref[...], preferred_element_type=jnp.float32)
    b = jnp.dot(h_neigh, wn2_ref[...], preferred_element_type=jnp.float32)
    out_ref[...] = jnp.maximum(a + b + bn_ref[...], 0.0)


def _finish(nf, partials, wn1, wn2, bn2):
    return pl.pallas_call(
        _final_body,
        out_shape=jax.ShapeDtypeStruct((N_NODES, HIDDEN), jnp.float32),
    )(nf, partials, wn1, wn2, bn2)


@jax.jit
def kernel(node_features, edge_index, static_edge_features, We, be, Wn, bn):
    src = edge_index[0].astype(jnp.int32)
    dst = edge_index[1].astype(jnp.int32)
    wet = We.T
    be2 = be.reshape(1, HIDDEN * NODE_IN)
    wn1 = Wn[:, :NODE_IN].T
    wn2 = Wn[:, NODE_IN:].T
    bn2 = bn.reshape(1, HIDDEN)
    zeros = jnp.zeros((GRP, MW), jnp.float32)
    rowidx = (jnp.arange(16, dtype=jnp.int32)[:, None] * ROWS_PER_TILE
              + jnp.arange(ROWS_PER_TILE, dtype=jnp.int32)
              ).reshape(16, RCH, GRP)
    dst2 = dst.reshape(N_GROUPS, GRP)

    h_src = _sc_gather(node_features, src)
    m_ext = _edge_messages(static_edge_features, h_src, wet, be2)
    partials = _sc_scatter(m_ext, dst2, rowidx, zeros)
    return _finish(node_features, partials, wn1, wn2, bn2)
